# baseline (speedup 1.0000x reference)
import jax
import jax.numpy as jnp
from jax.experimental import pallas as pl
from jax.experimental.pallas import tpu as pltpu
from jax.experimental.pallas import tpu_sc as plsc

_B = 64
_N = 1_000_000
_K = 256
_CHUNK = 1024
_WOUT = 65536
_L = 16
_NC = 2
_NS = 16
_NW = _NC * _NS
_RPW = _B // _NW


def _cumsum_lanes(x):
    n = x.shape[1]
    zeros_cache = {}
    s = 1
    while s < n:
        if s not in zeros_cache:
            zeros_cache[s] = jnp.zeros((x.shape[0], s), x.dtype)
        x = x + jnp.concatenate([zeros_cache[s], x[:, : n - s]], axis=1)
        s *= 2
    return x


def _sc_scan_body(x_hbm, sv_hbm, idx_hbm, xbuf, svbuf, outbuf):
    c = jax.lax.axis_index("c")
    s = jax.lax.axis_index("s")
    w = s * _NC + c
    for r in range(_RPW):
        row = w * _RPW + r
        pltpu.sync_copy(x_hbm.at[row, pl.ds(0, _K)], xbuf)
        pltpu.sync_copy(sv_hbm.at[row], svbuf)
        sv_vec = svbuf[...]

        def body(i, carry):
            carry_vec, cnt_vec = carry
            ics = carry_vec + plsc.cumsum(xbuf[pl.ds(i * _L, _L)])
            cnt_vec = cnt_vec + plsc.all_reduce_population_count(ics < sv_vec)
            carry_vec = jnp.zeros((_L,), jnp.float32) + jnp.max(ics)
            return carry_vec, cnt_vec

        _, cnt_vec = jax.lax.fori_loop(
            0, _K // _L, body,
            (jnp.zeros((_L,), jnp.float32), jnp.zeros((_L,), jnp.int32)))
        outbuf[...] = cnt_vec
        pltpu.sync_copy(outbuf, idx_hbm.at[row])


def _sc_scan(inputs, sv):
    mesh = plsc.VectorSubcoreMesh(core_axis_name="c", subcore_axis_name="s",
                                  num_cores=_NC, num_subcores=_NS)
    sv_b = jnp.broadcast_to(sv, (_B, _L))
    fn = pl.kernel(
        _sc_scan_body,
        out_type=jax.ShapeDtypeStruct((_B, _L), jnp.int32),
        mesh=mesh,
        compiler_params=pltpu.CompilerParams(needs_layout_passes=False),
        scratch_types=[
            pltpu.VMEM((_K,), jnp.float32),
            pltpu.VMEM((_L,), jnp.float32),
            pltpu.VMEM((_L,), jnp.int32),
        ],
    )
    return fn(inputs, sv_b)


def _full_scan_kernel(x_ref, sv_ref, idx_ref, carry_ref, acc_ref, done_ref):
    k = pl.program_id(0)

    @pl.when(k == 0)
    def _init():
        carry_ref[...] = jnp.zeros_like(carry_ref)
        acc_ref[...] = jnp.zeros_like(acc_ref)
        done_ref[...] = jnp.zeros_like(done_ref)

    col = k * _CHUNK + jax.lax.broadcasted_iota(jnp.int32, (_B, _CHUNK), 1)
    valid = col < _N
    x = jnp.where(valid, x_ref[...], 0.0)
    ics = carry_ref[...] + _cumsum_lanes(x)
    lt = jnp.logical_and(ics < sv_ref[...], valid)
    cnt = jnp.sum(lt.astype(jnp.int32), axis=1, keepdims=True)
    nvalid = jnp.sum(valid.astype(jnp.int32), axis=1, keepdims=True)
    done = done_ref[...]
    acc_ref[...] = acc_ref[...] + jnp.where(done > 0, 0, cnt)
    done_ref[...] = jnp.maximum(done, (cnt < nvalid).astype(jnp.int32))
    carry_ref[...] = ics[:, _CHUNK - 1:_CHUNK]

    @pl.when(k == pl.num_programs(0) - 1)
    def _emit():
        idx_ref[...] = acc_ref[...]


def _zeros_kernel(o_ref):
    o_ref[...] = jnp.zeros(o_ref.shape, jnp.float32)


_WSC = 256


def _fast_patch_kernel(idx_ref, zeros_ref, o_ref, stage_ref, sem):
    del zeros_ref
    cols = jax.lax.broadcasted_iota(jnp.int32, (_B, _WSC), 1)
    stage_ref[...] = (cols == idx_ref[...]).astype(jnp.float32)
    cp = pltpu.make_async_copy(stage_ref, o_ref.at[:, pl.ds(0, _WSC)], sem)
    cp.start()
    cp.wait()


def _onehot_kernel(idx_ref, o_ref):
    j = pl.program_id(0)
    col = j * _WOUT + jax.lax.broadcasted_iota(jnp.int32, o_ref.shape, 1)
    o_ref[...] = (col == idx_ref[...]).astype(jnp.float32)


def _full_scan(inputs, sv):
    return pl.pallas_call(
        _full_scan_kernel,
        grid=(pl.cdiv(_N, _CHUNK),),
        in_specs=[
            pl.BlockSpec((_B, _CHUNK), lambda k: (0, k)),
            pl.BlockSpec((_B, 1), lambda k: (0, 0)),
        ],
        out_specs=pl.BlockSpec((_B, 1), lambda k: (0, 0)),
        out_shape=jax.ShapeDtypeStruct((_B, 1), jnp.int32),
        scratch_shapes=[
            pltpu.VMEM((_B, 1), jnp.float32),
            pltpu.VMEM((_B, 1), jnp.int32),
            pltpu.VMEM((_B, 1), jnp.int32),
        ],
    )(inputs, sv)


def kernel(inputs):
    skey = jax.random.fold_in(jax.random.key(0), 1)
    sv = jax.random.uniform(skey, (_B, 1), dtype=inputs.dtype,
                            minval=0.0, maxval=1.0)

    idx16 = _sc_scan(inputs, sv)
    idx0 = idx16[:, :1]

    zeros = pl.pallas_call(
        _zeros_kernel,
        grid=(pl.cdiv(_N, _WOUT),),
        out_specs=pl.BlockSpec((_B, _WOUT), lambda j: (0, j)),
        out_shape=jax.ShapeDtypeStruct((_B, _N), jnp.float32),
    )()

    def _fast():
        return pl.pallas_call(
            _fast_patch_kernel,
            in_specs=[
                pl.BlockSpec((_B, 1), lambda: (0, 0)),
                pl.BlockSpec(memory_space=pl.ANY),
            ],
            out_specs=pl.BlockSpec(memory_space=pl.ANY),
            out_shape=jax.ShapeDtypeStruct((_B, _N), jnp.float32),
            scratch_shapes=[
                pltpu.VMEM((_B, _WSC), jnp.float32),
                pltpu.SemaphoreType.DMA,
            ],
            input_output_aliases={1: 0},
        )(idx0, zeros)

    def _slow():
        idx = _full_scan(inputs, sv)
        return pl.pallas_call(
            _onehot_kernel,
            grid=(pl.cdiv(_N, _WOUT),),
            in_specs=[pl.BlockSpec((_B, 1), lambda j: (0, 0))],
            out_specs=pl.BlockSpec((_B, _WOUT), lambda j: (0, j)),
            out_shape=jax.ShapeDtypeStruct((_B, _N), jnp.float32),
        )(idx)

    return jax.lax.cond(jnp.all(idx0 < _K), _fast, _slow)


# speedup vs baseline: 1.0146x; 1.0146x over previous
import jax
import jax.numpy as jnp
from jax.experimental import pallas as pl
from jax.experimental.pallas import tpu as pltpu
from jax.experimental.pallas import tpu_sc as plsc

_B = 64
_N = 1_000_000
_K = 256
_CHUNK = 1024
_WOUT = 65536
_L = 16
_NC = 1
_NS = 16
_NW = _NC * _NS
_RPW = _B // _NW


def _cumsum_lanes(x):
    n = x.shape[1]
    zeros_cache = {}
    s = 1
    while s < n:
        if s not in zeros_cache:
            zeros_cache[s] = jnp.zeros((x.shape[0], s), x.dtype)
        x = x + jnp.concatenate([zeros_cache[s], x[:, : n - s]], axis=1)
        s *= 2
    return x


def _sc_scan_body(x_hbm, sv_hbm, idx_hbm, xbuf, svbuf, outbuf, sem):
    c = jax.lax.axis_index("c")
    s = jax.lax.axis_index("s")
    w = s * _NC + c
    row0 = w * _RPW
    del sem
    for r in range(_RPW):
        pltpu.sync_copy(x_hbm.at[row0 + r, pl.ds(0, _K)],
                        xbuf.at[r])
        pltpu.sync_copy(sv_hbm.at[row0 + r], svbuf.at[r])
    for r in range(_RPW):
        sv_vec = svbuf[r]

        def body(i, carry):
            carry_vec, cnt_vec = carry
            ics = carry_vec + plsc.cumsum(xbuf[r, pl.ds(i * _L, _L)])
            cnt_vec = cnt_vec + plsc.all_reduce_population_count(ics < sv_vec)
            carry_vec = jnp.zeros((_L,), jnp.float32) + jnp.max(ics)
            return carry_vec, cnt_vec

        _, cnt_vec = jax.lax.fori_loop(
            0, _K // _L, body,
            (jnp.zeros((_L,), jnp.float32), jnp.zeros((_L,), jnp.int32)))
        outbuf[r] = cnt_vec
    pltpu.sync_copy(outbuf, idx_hbm.at[pl.ds(row0, _RPW)])


def _sc_scan(inputs, sv):
    mesh = plsc.VectorSubcoreMesh(core_axis_name="c", subcore_axis_name="s",
                                  num_cores=_NC, num_subcores=_NS)
    sv_b = jnp.broadcast_to(sv, (_B, _L))
    fn = pl.kernel(
        _sc_scan_body,
        out_type=jax.ShapeDtypeStruct((_B, _L), jnp.int32),
        mesh=mesh,
        compiler_params=pltpu.CompilerParams(needs_layout_passes=False),
        scratch_types=[
            pltpu.VMEM((_RPW, _K), jnp.float32),
            pltpu.VMEM((_RPW, _L), jnp.float32),
            pltpu.VMEM((_RPW, _L), jnp.int32),
            pltpu.SemaphoreType.DMA,
        ],
    )
    return fn(inputs, sv_b)


def _full_scan_kernel(x_ref, sv_ref, idx_ref, carry_ref, acc_ref, done_ref):
    k = pl.program_id(0)

    @pl.when(k == 0)
    def _init():
        carry_ref[...] = jnp.zeros_like(carry_ref)
        acc_ref[...] = jnp.zeros_like(acc_ref)
        done_ref[...] = jnp.zeros_like(done_ref)

    col = k * _CHUNK + jax.lax.broadcasted_iota(jnp.int32, (_B, _CHUNK), 1)
    valid = col < _N
    x = jnp.where(valid, x_ref[...], 0.0)
    ics = carry_ref[...] + _cumsum_lanes(x)
    lt = jnp.logical_and(ics < sv_ref[...], valid)
    cnt = jnp.sum(lt.astype(jnp.int32), axis=1, keepdims=True)
    nvalid = jnp.sum(valid.astype(jnp.int32), axis=1, keepdims=True)
    done = done_ref[...]
    acc_ref[...] = acc_ref[...] + jnp.where(done > 0, 0, cnt)
    done_ref[...] = jnp.maximum(done, (cnt < nvalid).astype(jnp.int32))
    carry_ref[...] = ics[:, _CHUNK - 1:_CHUNK]

    @pl.when(k == pl.num_programs(0) - 1)
    def _emit():
        idx_ref[...] = acc_ref[...]


def _zeros_kernel(o_ref):
    o_ref[...] = jnp.zeros(o_ref.shape, jnp.float32)


_WSC = 256


def _fast_patch_kernel(idx_ref, zeros_ref, o_ref):
    del zeros_ref
    cols = jax.lax.broadcasted_iota(jnp.int32, (_B, _WSC), 1)
    o_ref[...] = (cols == idx_ref[...]).astype(jnp.float32)


def _onehot_kernel(idx_ref, o_ref):
    j = pl.program_id(0)
    col = j * _WOUT + jax.lax.broadcasted_iota(jnp.int32, o_ref.shape, 1)
    o_ref[...] = (col == idx_ref[...]).astype(jnp.float32)


def _full_scan(inputs, sv):
    return pl.pallas_call(
        _full_scan_kernel,
        grid=(pl.cdiv(_N, _CHUNK),),
        in_specs=[
            pl.BlockSpec((_B, _CHUNK), lambda k: (0, k)),
            pl.BlockSpec((_B, 1), lambda k: (0, 0)),
        ],
        out_specs=pl.BlockSpec((_B, 1), lambda k: (0, 0)),
        out_shape=jax.ShapeDtypeStruct((_B, 1), jnp.int32),
        scratch_shapes=[
            pltpu.VMEM((_B, 1), jnp.float32),
            pltpu.VMEM((_B, 1), jnp.int32),
            pltpu.VMEM((_B, 1), jnp.int32),
        ],
    )(inputs, sv)


def kernel(inputs):
    skey = jax.random.fold_in(jax.random.key(0), 1)
    sv = jax.random.uniform(skey, (_B, 1), dtype=inputs.dtype,
                            minval=0.0, maxval=1.0)

    idx16 = _sc_scan(inputs, sv)
    idx0 = idx16[:, :1]

    zeros = pl.pallas_call(
        _zeros_kernel,
        grid=(pl.cdiv(_N, _WOUT),),
        out_specs=pl.BlockSpec((_B, _WOUT), lambda j: (0, j)),
        out_shape=jax.ShapeDtypeStruct((_B, _N), jnp.float32),
    )()

    def _fast():
        return pl.pallas_call(
            _fast_patch_kernel,
            grid=(1,),
            in_specs=[
                pl.BlockSpec((_B, 1), lambda i: (0, 0)),
                pl.BlockSpec(memory_space=pl.ANY),
            ],
            out_specs=pl.BlockSpec((_B, _WSC), lambda i: (0, 0)),
            out_shape=jax.ShapeDtypeStruct((_B, _N), jnp.float32),
            input_output_aliases={1: 0},
        )(idx0, zeros)

    def _slow():
        idx = _full_scan(inputs, sv)
        return pl.pallas_call(
            _onehot_kernel,
            grid=(pl.cdiv(_N, _WOUT),),
            in_specs=[pl.BlockSpec((_B, 1), lambda j: (0, 0))],
            out_specs=pl.BlockSpec((_B, _WOUT), lambda j: (0, j)),
            out_shape=jax.ShapeDtypeStruct((_B, _N), jnp.float32),
        )(idx)

    return jax.lax.cond(jnp.all(idx0 < _K), _fast, _slow)
